# baseline (speedup 1.0000x reference)
import functools

import jax
import jax.numpy as jnp
from jax import lax
from jax.experimental import pallas as pl
from jax.experimental.pallas import tpu as pltpu
from jax.experimental.pallas import tpu_sc as plsc

N = 10000
E = 320000
D = 128
D_OUT = 128

NC = 2
NS = 16
CHUNK = 80
EDGES_PER_CORE = E // NC
EDGES_PER_SUB = EDGES_PER_CORE // NS
NCHUNKS = EDGES_PER_SUB // CHUNK
N_PAD = 10240
ROWS_PER_SUB = N_PAD // NS

ROW_BLK = 1000


NBUF = 4
IBUF = 8
GLAG = 2
DLAG = 4
SLAG = 6
NROUNDS = -(-(NCHUNKS + SLAG) // IBUF)


def _seg_sum_body(x_hbm, ei_hbm, zeros_hbm, out_hbm, acc, *sc):
    ibuf = sc[0:IBUF]
    rows = sc[IBUF:IBUF + NBUF]
    isem = sc[IBUF + NBUF:2 * IBUF + NBUF]
    gsem = sc[2 * IBUF + NBUF:2 * IBUF + 2 * NBUF]
    ssem = sc[2 * IBUF + 2 * NBUF:2 * IBUF + 3 * NBUF]
    c = lax.axis_index("c")
    s = lax.axis_index("s")
    w = c * NS + s

    pltpu.sync_copy(zeros_hbm, acc.at[pl.ds(s * ROWS_PER_SUB, ROWS_PER_SUB)])
    plsc.subcore_barrier()

    @pl.loop(0, NROUNDS)
    def _round(r):
        for b in range(IBUF):
            i = r * IBUF + b

            @pl.when(jnp.logical_and(i >= SLAG, i < NCHUNKS + SLAG))
            def _drain_scatter():
                pltpu.make_async_copy(
                    rows[(b - SLAG) % NBUF],
                    acc.at[ibuf[(b - SLAG) % IBUF].at[1]],
                    ssem[(b - SLAG) % NBUF]).wait()

            @pl.when(i < NCHUNKS)
            def _load_idx():
                pltpu.async_copy(ei_hbm.at[w, i], ibuf[b], isem[b])

            @pl.when(jnp.logical_and(i >= GLAG, i < NCHUNKS + GLAG))
            def _issue_gather():
                pltpu.make_async_copy(
                    ei_hbm.at[w, i - GLAG], ibuf[(b - GLAG) % IBUF],
                    isem[(b - GLAG) % IBUF]).wait()
                pltpu.async_copy(x_hbm.at[ibuf[(b - GLAG) % IBUF].at[0]],
                                 rows[(b - GLAG) % NBUF],
                                 gsem[(b - GLAG) % NBUF])

            @pl.when(jnp.logical_and(i >= DLAG, i < NCHUNKS + DLAG))
            def _issue_scatter():
                pltpu.make_async_copy(
                    x_hbm.at[ibuf[(b - DLAG) % IBUF].at[0]],
                    rows[(b - DLAG) % NBUF],
                    gsem[(b - DLAG) % NBUF]).wait()
                pltpu.async_copy(rows[(b - DLAG) % NBUF],
                                 acc.at[ibuf[(b - DLAG) % IBUF].at[1]],
                                 ssem[(b - DLAG) % NBUF], add=True)

    plsc.subcore_barrier()
    pltpu.sync_copy(acc.at[pl.ds(s * ROWS_PER_SUB, ROWS_PER_SUB)],
                    out_hbm.at[c, pl.ds(s * ROWS_PER_SUB, ROWS_PER_SUB)])


def _segment_sum_sc(x, ei4d, zeros):
    mesh = plsc.VectorSubcoreMesh(core_axis_name="c", subcore_axis_name="s")
    k = pl.kernel(
        _seg_sum_body,
        out_type=jax.ShapeDtypeStruct((NC, N_PAD, D), jnp.float32),
        mesh=mesh,
        scratch_types=[pltpu.VMEM_SHARED((N_PAD, D), jnp.float32)]
        + [pltpu.VMEM((2, CHUNK), jnp.int32) for _ in range(IBUF)]
        + [pltpu.VMEM((CHUNK, D), jnp.float32) for _ in range(NBUF)]
        + [pltpu.SemaphoreType.DMA for _ in range(IBUF + 2 * NBUF)],
    )
    return k(x, ei4d, zeros)


def _prep_body(w2_ref, wh_ref, b2_ref, bh_ref, m_ref, bh2_ref):
    m_ref[...] = jnp.dot(w2_ref[...], wh_ref[...],
                         preferred_element_type=jnp.float32)
    bh2_ref[...] = (jnp.dot(b2_ref[...], wh_ref[...],
                            preferred_element_type=jnp.float32) + bh_ref[...])


def _prep(W2, Wh, b2, bh):
    return pl.pallas_call(
        _prep_body,
        out_shape=(jax.ShapeDtypeStruct((D, D_OUT), jnp.float32),
                   jax.ShapeDtypeStruct((1, D_OUT), jnp.float32)),
    )(W2, Wh, b2.reshape(1, D), bh.reshape(1, D_OUT))


def _xw_body(x_ref, w_ref, o_ref):
    o_ref[...] = jnp.dot(x_ref[...], w_ref[...],
                         preferred_element_type=jnp.float32)


def _xw(x, W):
    return pl.pallas_call(
        _xw_body,
        out_shape=jax.ShapeDtypeStruct((N, D_OUT), jnp.float32),
    )(x, W)


def _comb_body(scale_ref, p_ref, agg_ref, w_ref, b_ref, o_ref):
    a = agg_ref[0, :N, :] + agg_ref[1, :N, :]
    o_ref[...] = (scale_ref[0, 0] * p_ref[...]
                  + jnp.dot(a, w_ref[...], preferred_element_type=jnp.float32)
                  + b_ref[...])


def _comb(eps, p, agg, W, b):
    scale = (1.0 + eps).astype(jnp.float32).reshape(1, 1)
    return pl.pallas_call(
        _comb_body,
        in_specs=[
            pl.BlockSpec(memory_space=pltpu.SMEM),
            pl.BlockSpec((N, D_OUT), lambda: (0, 0)),
            pl.BlockSpec((NC, N_PAD, D), lambda: (0, 0, 0)),
            pl.BlockSpec((D, D_OUT), lambda: (0, 0)),
            pl.BlockSpec((1, D_OUT), lambda: (0, 0)),
        ],
        out_specs=pl.BlockSpec((N, D_OUT), lambda: (0, 0)),
        out_shape=jax.ShapeDtypeStruct((N, D_OUT), jnp.float32),
    )(scale, p, agg, W, b)


def kernel(x, edge_index, W1, b1, eps1, W2, b2, eps2, Wh, bh):
    ei4d = edge_index.reshape(2, NC * NS, NCHUNKS, CHUNK).transpose(1, 2, 0, 3)
    zeros = jnp.zeros((ROWS_PER_SUB, D), jnp.float32)
    M, bh2 = _prep(W2, Wh, b2, bh)
    agg1 = _segment_sum_sc(x, ei4d, zeros)
    P = _xw(x, W1)
    h = _comb(eps1, P, agg1, W1, b1.reshape(1, D))
    agg2 = _segment_sum_sc(h, ei4d, zeros)
    Q = _xw(h, M)
    return _comb(eps2, Q, agg2, M, bh2)


# speedup vs baseline: 1.0154x; 1.0154x over previous
import functools

import jax
import jax.numpy as jnp
from jax import lax
from jax.experimental import pallas as pl
from jax.experimental.pallas import tpu as pltpu
from jax.experimental.pallas import tpu_sc as plsc

N = 10000
E = 320000
D = 128
D_OUT = 128

NC = 2
NS = 16
CHUNK = 80
EDGES_PER_CORE = E // NC
EDGES_PER_SUB = EDGES_PER_CORE // NS
NCHUNKS = EDGES_PER_SUB // CHUNK
N_PAD = 10240
ROWS_PER_SUB = N_PAD // NS

ROW_BLK = 1000


NBUF = 4
IBUF = 8
GLAG = 2
DLAG = 4
SLAG = 6
NROUNDS = -(-(NCHUNKS + SLAG) // IBUF)


def _seg_sum_body(x_hbm, ei_hbm, zeros_hbm, out_hbm, acc, *sc):
    ibuf = sc[0:IBUF]
    rows = sc[IBUF:IBUF + NBUF]
    isem = sc[IBUF + NBUF:2 * IBUF + NBUF]
    gsem = sc[2 * IBUF + NBUF:2 * IBUF + 2 * NBUF]
    ssem = sc[2 * IBUF + 2 * NBUF:2 * IBUF + 3 * NBUF]
    c = lax.axis_index("c")
    s = lax.axis_index("s")
    w = c * NS + s

    pltpu.sync_copy(zeros_hbm, acc.at[pl.ds(s * ROWS_PER_SUB, ROWS_PER_SUB)])
    plsc.subcore_barrier()

    @pl.loop(0, NROUNDS)
    def _round(r):
        for b in range(IBUF):
            i = r * IBUF + b

            @pl.when(jnp.logical_and(i >= SLAG, i < NCHUNKS + SLAG))
            def _drain_scatter():
                pltpu.make_async_copy(
                    rows[(b - SLAG) % NBUF],
                    acc.at[ibuf[(b - SLAG) % IBUF].at[1]],
                    ssem[(b - SLAG) % NBUF]).wait()

            @pl.when(i < NCHUNKS)
            def _load_idx():
                pltpu.async_copy(ei_hbm.at[w, i], ibuf[b], isem[b])

            @pl.when(jnp.logical_and(i >= GLAG, i < NCHUNKS + GLAG))
            def _issue_gather():
                pltpu.make_async_copy(
                    ei_hbm.at[w, i - GLAG], ibuf[(b - GLAG) % IBUF],
                    isem[(b - GLAG) % IBUF]).wait()
                pltpu.async_copy(x_hbm.at[ibuf[(b - GLAG) % IBUF].at[0]],
                                 rows[(b - GLAG) % NBUF],
                                 gsem[(b - GLAG) % NBUF])

            @pl.when(jnp.logical_and(i >= DLAG, i < NCHUNKS + DLAG))
            def _issue_scatter():
                pltpu.make_async_copy(
                    x_hbm.at[ibuf[(b - DLAG) % IBUF].at[0]],
                    rows[(b - DLAG) % NBUF],
                    gsem[(b - DLAG) % NBUF]).wait()
                pltpu.async_copy(rows[(b - DLAG) % NBUF],
                                 acc.at[ibuf[(b - DLAG) % IBUF].at[1]],
                                 ssem[(b - DLAG) % NBUF], add=True)

    plsc.subcore_barrier()
    pltpu.sync_copy(acc.at[pl.ds(s * ROWS_PER_SUB, ROWS_PER_SUB)],
                    out_hbm.at[c, pl.ds(s * ROWS_PER_SUB, ROWS_PER_SUB)])


def _segment_sum_sc(x, ei4d, zeros):
    mesh = plsc.VectorSubcoreMesh(core_axis_name="c", subcore_axis_name="s")
    k = pl.kernel(
        _seg_sum_body,
        out_type=jax.ShapeDtypeStruct((NC, N_PAD, D), jnp.float32),
        mesh=mesh,
        scratch_types=[pltpu.VMEM_SHARED((N_PAD, D), jnp.float32)]
        + [pltpu.VMEM((2, CHUNK), jnp.int32) for _ in range(IBUF)]
        + [pltpu.VMEM((CHUNK, D), jnp.float32) for _ in range(NBUF)]
        + [pltpu.SemaphoreType.DMA for _ in range(IBUF + 2 * NBUF)],
    )
    return k(x, ei4d, zeros)


def _mlp1_body(scale_ref, x_ref, agg_ref, w_ref, b_ref, o_ref):
    t = scale_ref[0, 0] * x_ref[...] + agg_ref[0, :N, :] + agg_ref[1, :N, :]
    o_ref[...] = (jnp.dot(t, w_ref[...], preferred_element_type=jnp.float32)
                  + b_ref[...])


def _mlp2_body(scale_ref, h_ref, agg_ref, w2_ref, b2_ref, wh_ref, bh_ref,
               o_ref):
    m = jnp.dot(w2_ref[...], wh_ref[...], preferred_element_type=jnp.float32)
    bh2 = (jnp.dot(b2_ref[...], wh_ref[...],
                   preferred_element_type=jnp.float32) + bh_ref[...])
    t = scale_ref[0, 0] * h_ref[...] + agg_ref[0, :N, :] + agg_ref[1, :N, :]
    o_ref[...] = jnp.dot(t, m, preferred_element_type=jnp.float32) + bh2


def _mlp1(x, agg, W1, b1, eps1):
    scale = (1.0 + eps1).astype(jnp.float32).reshape(1, 1)
    return pl.pallas_call(
        _mlp1_body,
        in_specs=[
            pl.BlockSpec(memory_space=pltpu.SMEM),
            pl.BlockSpec((N, D), lambda: (0, 0)),
            pl.BlockSpec((NC, N_PAD, D), lambda: (0, 0, 0)),
            pl.BlockSpec((D, D), lambda: (0, 0)),
            pl.BlockSpec((1, D), lambda: (0, 0)),
        ],
        out_specs=pl.BlockSpec((N, D), lambda: (0, 0)),
        out_shape=jax.ShapeDtypeStruct((N, D), jnp.float32),
    )(scale, x, agg, W1, b1.reshape(1, D))


def _mlp2_head(h, agg, W2, b2, eps2, Wh, bh):
    scale = (1.0 + eps2).astype(jnp.float32).reshape(1, 1)
    return pl.pallas_call(
        _mlp2_body,
        in_specs=[
            pl.BlockSpec(memory_space=pltpu.SMEM),
            pl.BlockSpec((N, D), lambda: (0, 0)),
            pl.BlockSpec((NC, N_PAD, D), lambda: (0, 0, 0)),
            pl.BlockSpec((D, D), lambda: (0, 0)),
            pl.BlockSpec((1, D), lambda: (0, 0)),
            pl.BlockSpec((D, D_OUT), lambda: (0, 0)),
            pl.BlockSpec((1, D_OUT), lambda: (0, 0)),
        ],
        out_specs=pl.BlockSpec((N, D_OUT), lambda: (0, 0)),
        out_shape=jax.ShapeDtypeStruct((N, D_OUT), jnp.float32),
    )(scale, h, agg, W2, b2.reshape(1, D), Wh, bh.reshape(1, D_OUT))


def kernel(x, edge_index, W1, b1, eps1, W2, b2, eps2, Wh, bh):
    ei4d = edge_index.reshape(2, NC * NS, NCHUNKS, CHUNK).transpose(1, 2, 0, 3)
    zeros = jnp.zeros((ROWS_PER_SUB, D), jnp.float32)
    agg1 = _segment_sum_sc(x, ei4d, zeros)
    h = _mlp1(x, agg1, W1, b1, eps1)
    agg2 = _segment_sum_sc(h, ei4d, zeros)
    return _mlp2_head(h, agg2, W2, b2, eps2, Wh, bh)
